# baseline (speedup 1.0000x reference)
import functools

import jax
import jax.numpy as jnp
from jax import lax
from jax.experimental import pallas as pl
from jax.experimental.pallas import tpu as pltpu
from jax.experimental.pallas import tpu_sc as plsc

D = 32

_info = plsc.get_sparse_core_info()
_NC, _NS = _info.num_cores, _info.num_subcores
NW = _NC * _NS


@functools.lru_cache(maxsize=None)
def _make_gather(H, B):
    assert B % (128 * NW) == 0, (H, B)
    CB = B // (128 * NW)
    BW = 128 * CB
    mesh = plsc.VectorSubcoreMesh(core_axis_name="c", subcore_axis_name="s")

    @functools.partial(
        pl.kernel,
        out_type=jax.ShapeDtypeStruct((H, D // 8, B // 128, 8, 128),
                                      jnp.float32),
        mesh=mesh,
        scratch_types=[
            pltpu.VMEM((2, BW), jnp.int32),
            pltpu.VMEM((2, CB, 128, D), jnp.float32),
            pltpu.SemaphoreType.DMA((2,)),
            pltpu.SemaphoreType.DMA((2,)),
            pltpu.SemaphoreType.DMA((2,)),
        ],
        compiler_params=pltpu.CompilerParams(use_tc_tiling_on_sc=False),
    )
    def gather_kernel(ids_hbm, table_hbm, out_hbm,
                      idx_v, g_v, sem_i, sem_g, sem_t):
        wid = lax.axis_index("s") * _NC + lax.axis_index("c")
        col0 = wid * BW
        cb0 = wid * CB

        def start_idx(h):
            s = lax.rem(h, 2)
            pltpu.async_copy(ids_hbm.at[h, pl.ds(col0, BW)],
                             idx_v.at[s], sem_i.at[s])

        def wait_idx(h):
            s = lax.rem(h, 2)
            pltpu.make_async_copy(ids_hbm.at[0, pl.ds(col0, BW)],
                                  idx_v.at[s], sem_i.at[s]).wait()

        def start_gathers(h):
            s = lax.rem(h, 2)
            for j in range(CB):
                pltpu.async_copy(
                    table_hbm.at[idx_v.at[s, pl.ds(128 * j, 128)]],
                    g_v.at[s, j], sem_g.at[s])

        def wait_gathers(h):
            s = lax.rem(h, 2)
            for j in range(CB):
                pltpu.make_async_copy(
                    table_hbm.at[idx_v.at[s, pl.ds(128 * j, 128)]],
                    g_v.at[s, j], sem_g.at[s]).wait()

        def start_stores(h):
            s = lax.rem(h, 2)
            for d in range(D):
                pltpu.async_copy(
                    g_v.at[s, :, :, d],
                    out_hbm.at[h, d // 8, pl.ds(cb0, CB), d % 8, :],
                    sem_t.at[s])

        def drain_stores(h):
            s = lax.rem(h, 2)
            for d in range(D):
                pltpu.make_async_copy(
                    g_v.at[s, :, :, d],
                    out_hbm.at[0, d // 8, pl.ds(cb0, CB), d % 8, :],
                    sem_t.at[s]).wait()

        pltpu.sync_copy(ids_hbm.at[0, pl.ds(col0, BW)], idx_v.at[0])
        start_gathers(0)
        start_idx(1)

        def body(h, carry):
            @pl.when(h + 2 < H)
            def _():
                start_idx(h + 2)

            @pl.when(h + 1 < H)
            def _():
                wait_idx(h + 1)

                @pl.when(h >= 1)
                def _():
                    drain_stores(h - 1)

                start_gathers(h + 1)

            wait_gathers(h)
            start_stores(h)
            return carry

        lax.fori_loop(0, H, body, 0)

        drain_stores(H - 2)
        drain_stores(H - 1)

    return gather_kernel


@jax.jit
def kernel(token_ids, weight):
    b, h = token_ids.shape
    ids_t = token_ids.T.astype(jnp.int32)
    o = _make_gather(h, b)(ids_t, weight)
    return o.transpose(2, 4, 0, 1, 3).reshape(b, h, weight.shape[1])
